# baseline (speedup 1.0000x reference)
import functools

import jax
import jax.numpy as jnp
from jax import lax
from jax.experimental import pallas as pl
from jax.experimental.pallas import tpu as pltpu
from jax.experimental.pallas import tpu_sc as plsc

N = 10000
D = 128
B = 500
EPT = 32
ZSTRIDE = 624
ZROWS = 64
ZFILL = 640
NSLOTS = 512
LANES = 16

_mesh = plsc.VectorSubcoreMesh(core_axis_name="c", subcore_axis_name="s")

_f32 = jnp.float32
_i32 = jnp.int32


def _body(uf, nf, efh, ei, out_u, out_n,
          zbuf, rowsv, efv, sumsv, featv, outv, eiv, idxo, idxp, cntv, onesv,
          acc, cnt_sh, s_zero, s_a, s_b, s_c):
    c = lax.axis_index("c")
    s = lax.axis_index("s")
    base = s * EPT

    is0 = c == 0
    is1 = c == 1

    dei = pltpu.async_copy(ei.at[pl.ds(0, 2), pl.ds(0, NSLOTS)], eiv, s_c)

    def _zb(i, t):
        for j in range(D // LANES):
            zbuf[i, pl.ds(LANES * j, LANES)] = jnp.zeros((LANES,), _f32)
        return t
    lax.fori_loop(0, ZROWS, _zb, 0)
    row0 = s * ZSTRIDE

    def _fire_zero(out_ref, chunks):
        for off, nchunk in chunks:
            pltpu.async_copy(zbuf.at[pl.ds(0, nchunk)],
                             out_ref.at[pl.ds(row0 + off, nchunk)],
                             s_zero)

    _C0 = [(i * 64, 64) for i in range(10)]
    _C1 = [(i * 64, 64) for i in range(9)] + [(576, 32), (608, 32)]
    _SITES0 = (_C0[0:1], _C0[1:4], _C0[4:7], _C0[7:10])
    _SITES1 = (_C1[0:2], _C1[2:4], _C1[4:8], _C1[8:11])

    def _fire_zero_both(site):
        @pl.when(is0)
        def _():
            _fire_zero(out_u, _SITES0[site])

        @pl.when(is1)
        def _():
            _fire_zero(out_n, _SITES1[site])

    _fire_zero_both(0)
    dei.wait()

    def _extract(own_row, opp_row):
        for k in range(EPT // LANES):
            sl = pl.ds(LANES * k, LANES)
            idxo[sl] = eiv[own_row, pl.ds(base + LANES * k, LANES)]
            idxp[sl] = eiv[opp_row, pl.ds(base + LANES * k, LANES)]

    @pl.when(is0)
    def _():
        _extract(0, 1)

    @pl.when(is1)
    def _():
        _extract(1, 0)

    @pl.when(is0)
    def _():
        pltpu.async_copy(nf.at[idxp], rowsv, s_a)
        pltpu.async_copy(uf.at[idxo], featv, s_c)

    @pl.when(is1)
    def _():
        pltpu.async_copy(nf.at[idxo], featv, s_c)
        pltpu.async_copy(uf.at[idxp], rowsv, s_a)

    d2 = pltpu.async_copy(efh.at[pl.ds(base, EPT)], efv, s_a)
    z1 = pltpu.async_copy(zbuf.at[pl.ds(0, EPT)], acc.at[idxo], s_b)
    z2 = pltpu.async_copy(zbuf.at[0, pl.ds(0, EPT)], cnt_sh.at[idxo], s_b)
    _fire_zero_both(1)

    @pl.when(is0)
    def _():
        pltpu.make_async_copy(nf.at[idxp], rowsv, s_a).wait()

    @pl.when(is1)
    def _():
        pltpu.make_async_copy(uf.at[idxp.at[pl.ds(0, EPT // 2)]],
                              rowsv.at[pl.ds(0, EPT // 2)], s_a).wait()
        pltpu.make_async_copy(uf.at[idxp.at[pl.ds(EPT // 2, EPT // 2)]],
                              rowsv.at[pl.ds(EPT // 2, EPT // 2)], s_a).wait()

    d2.wait()
    z1.wait()
    z2.wait()
    plsc.subcore_barrier()

    def _msg(e, t):
        w = jnp.where(base + e < B, jnp.float32(1.0), jnp.float32(0.0))
        for j in range(D // LANES):
            sl = pl.ds(LANES * j, LANES)
            rowsv[e, sl] = rowsv[e, sl] * efv[e, sl] * w
        return t
    lax.fori_loop(0, EPT, _msg, 0)
    for k in range(EPT // LANES):
        lane = base + LANES * k + lax.iota(_i32, LANES)
        onesv[pl.ds(LANES * k, LANES)] = jnp.where(
            lane < B, jnp.float32(1.0), jnp.float32(0.0))

    a1 = pltpu.async_copy(rowsv, acc.at[idxo], s_b, add=True)
    a2 = pltpu.async_copy(onesv, cnt_sh.at[idxo], s_b, add=True)
    _fire_zero_both(2)
    a1.wait()
    a2.wait()
    plsc.subcore_barrier()

    d3 = pltpu.async_copy(acc.at[idxo], sumsv, s_a)
    d4 = pltpu.async_copy(cnt_sh.at[idxo], cntv, s_a)
    _fire_zero_both(3)
    d3.wait()
    d4.wait()

    @pl.when(is0)
    def _():
        pltpu.make_async_copy(uf.at[idxo], featv, s_c).wait()

    @pl.when(is1)
    def _():
        pltpu.make_async_copy(nf.at[idxo.at[pl.ds(0, EPT // 2)]],
                              featv.at[pl.ds(0, EPT // 2)], s_c).wait()
        pltpu.make_async_copy(nf.at[idxo.at[pl.ds(EPT // 2, EPT // 2)]],
                              featv.at[pl.ds(EPT // 2, EPT // 2)], s_c).wait()

    def _out(e, t):
        cvec = plsc.load_gather(cntv, [jnp.full((LANES,), e, _i32)])
        flag = (cvec > 0).astype(_f32)
        inv = jnp.float32(1.0) / jnp.maximum(cvec, jnp.float32(1.0))
        for j in range(D // LANES):
            sl = pl.ds(LANES * j, LANES)
            outv[e, sl] = (sumsv[e, sl] * inv + featv[e, sl]) * flag
        return t
    lax.fori_loop(0, EPT, _out, 0)

    def _drain_zero(out_ref, chunks):
        for _, nchunk in chunks:
            pltpu.make_async_copy(zbuf.at[pl.ds(0, nchunk)],
                                  out_ref.at[pl.ds(0, nchunk)], s_zero).wait()

    @pl.when(is0)
    def _():
        _drain_zero(out_u, _C0)

    @pl.when(is1)
    def _():
        _drain_zero(out_n, _C1)

    plsc.subcore_barrier()

    @pl.when(is0)
    def _():
        pltpu.sync_copy(outv, out_u.at[idxo])

    @pl.when(is1)
    def _():
        d9 = pltpu.async_copy(outv, out_n.at[idxo], s_c)
        d9.wait()


_gcn_sc = functools.partial(
    pl.kernel,
    out_type=(jax.ShapeDtypeStruct((N, D), _f32),
              jax.ShapeDtypeStruct((N, D), _f32)),
    mesh=_mesh,
    compiler_params=pltpu.CompilerParams(
        needs_layout_passes=False,
        skip_device_barrier=True,
        disable_bounds_checks=True,
        disable_semaphore_checks=True,
    ),
    scratch_types=[
        pltpu.VMEM((ZROWS, D), _f32),
        pltpu.VMEM((EPT, D), _f32),
        pltpu.VMEM((EPT, D), _f32),
        pltpu.VMEM((EPT, D), _f32),
        pltpu.VMEM((EPT, D), _f32),
        pltpu.VMEM((EPT, D), _f32),
        pltpu.VMEM((2, NSLOTS), _i32),
        pltpu.VMEM((EPT,), _i32),
        pltpu.VMEM((EPT,), _i32),
        pltpu.VMEM((EPT,), _f32),
        pltpu.VMEM((EPT,), _f32),
        pltpu.VMEM_SHARED((N, D), _f32),
        pltpu.VMEM_SHARED((N,), _f32),
        pltpu.SemaphoreType.DMA,
        pltpu.SemaphoreType.DMA,
        pltpu.SemaphoreType.DMA,
        pltpu.SemaphoreType.DMA,
    ],
)(_body)


def kernel(user_feat, news_feat, edge_feat, edge_index):
    return _gcn_sc(user_feat, news_feat, edge_feat, edge_index)


# speedup vs baseline: 1.0577x; 1.0577x over previous
import functools

import jax
import jax.numpy as jnp
from jax import lax
from jax.experimental import pallas as pl
from jax.experimental.pallas import tpu as pltpu
from jax.experimental.pallas import tpu_sc as plsc

N = 10000
D = 128
B = 500
EPT = 32
ZSTRIDE = 624
ZROWS = 64
ZFILL = 640
NSLOTS = 512
LANES = 16

_mesh = plsc.VectorSubcoreMesh(core_axis_name="c", subcore_axis_name="s")

_f32 = jnp.float32
_i32 = jnp.int32


def _body(uf, nf, efh, ei, out_u, out_n,
          zbuf, rowsv, efv, sumsv, featv, outv, eiv, idxo, idxp, cntv, onesv,
          acc, cnt_sh, s_zero, s_a, s_b, s_c):
    c = lax.axis_index("c")
    s = lax.axis_index("s")
    base = s * EPT

    is0 = c == 0
    is1 = c == 1

    dei = pltpu.async_copy(ei.at[pl.ds(0, 2), pl.ds(0, NSLOTS)], eiv, s_c)

    def _zb(i, t):
        for j in range(D // LANES):
            zbuf[i, pl.ds(LANES * j, LANES)] = jnp.zeros((LANES,), _f32)
        return t
    lax.fori_loop(0, ZROWS, _zb, 0)
    row0 = s * ZSTRIDE

    def _fire_zero(out_ref, chunks):
        for off, nchunk in chunks:
            pltpu.async_copy(zbuf.at[pl.ds(0, nchunk)],
                             out_ref.at[pl.ds(row0 + off, nchunk)],
                             s_zero)

    _C0 = [(i * 64, 64) for i in range(10)]
    _C1 = [(i * 64, 64) for i in range(9)] + [(576, 32), (608, 32)]
    _SITES0 = (_C0[0:2], _C0[2:6], _C0[6:10])
    _SITES1 = (_C1[0:3], _C1[3:7], _C1[7:11])

    def _fire_zero_both(site):
        @pl.when(is0)
        def _():
            _fire_zero(out_u, _SITES0[site])

        @pl.when(is1)
        def _():
            _fire_zero(out_n, _SITES1[site])

    dei.wait()

    def _extract(own_row, opp_row):
        for k in range(EPT // LANES):
            sl = pl.ds(LANES * k, LANES)
            idxo[sl] = eiv[own_row, pl.ds(base + LANES * k, LANES)]
            idxp[sl] = eiv[opp_row, pl.ds(base + LANES * k, LANES)]

    @pl.when(is0)
    def _():
        _extract(0, 1)

    @pl.when(is1)
    def _():
        _extract(1, 0)

    @pl.when(is0)
    def _():
        pltpu.async_copy(nf.at[idxp], rowsv, s_a)
        pltpu.async_copy(uf.at[idxo], featv, s_c)

    @pl.when(is1)
    def _():
        pltpu.async_copy(nf.at[idxo], featv, s_c)
        pltpu.async_copy(uf.at[idxp], rowsv, s_a)

    d2 = pltpu.async_copy(efh.at[pl.ds(base, EPT)], efv, s_a)
    z1 = pltpu.async_copy(zbuf.at[pl.ds(0, EPT)], acc.at[idxo], s_b)
    z2 = pltpu.async_copy(zbuf.at[0, pl.ds(0, EPT)], cnt_sh.at[idxo], s_b)
    _fire_zero_both(0)

    @pl.when(is0)
    def _():
        pltpu.make_async_copy(nf.at[idxp], rowsv, s_a).wait()

    @pl.when(is1)
    def _():
        pltpu.make_async_copy(uf.at[idxp.at[pl.ds(0, EPT // 2)]],
                              rowsv.at[pl.ds(0, EPT // 2)], s_a).wait()
        pltpu.make_async_copy(uf.at[idxp.at[pl.ds(EPT // 2, EPT // 2)]],
                              rowsv.at[pl.ds(EPT // 2, EPT // 2)], s_a).wait()

    d2.wait()
    z1.wait()
    z2.wait()
    plsc.subcore_barrier()

    def _msg(e, t):
        w = jnp.where(base + e < B, jnp.float32(1.0), jnp.float32(0.0))
        for j in range(D // LANES):
            sl = pl.ds(LANES * j, LANES)
            rowsv[e, sl] = rowsv[e, sl] * efv[e, sl] * w
        return t
    lax.fori_loop(0, EPT, _msg, 0)
    for k in range(EPT // LANES):
        lane = base + LANES * k + lax.iota(_i32, LANES)
        onesv[pl.ds(LANES * k, LANES)] = jnp.where(
            lane < B, jnp.float32(1.0), jnp.float32(0.0))

    a1 = pltpu.async_copy(rowsv, acc.at[idxo], s_b, add=True)
    a2 = pltpu.async_copy(onesv, cnt_sh.at[idxo], s_b, add=True)
    _fire_zero_both(1)
    a1.wait()
    a2.wait()
    plsc.subcore_barrier()

    d3 = pltpu.async_copy(acc.at[idxo], sumsv, s_a)
    d4 = pltpu.async_copy(cnt_sh.at[idxo], cntv, s_a)
    _fire_zero_both(2)
    d3.wait()
    d4.wait()

    @pl.when(is0)
    def _():
        pltpu.make_async_copy(uf.at[idxo], featv, s_c).wait()

    @pl.when(is1)
    def _():
        pltpu.make_async_copy(nf.at[idxo.at[pl.ds(0, EPT // 2)]],
                              featv.at[pl.ds(0, EPT // 2)], s_c).wait()
        pltpu.make_async_copy(nf.at[idxo.at[pl.ds(EPT // 2, EPT // 2)]],
                              featv.at[pl.ds(EPT // 2, EPT // 2)], s_c).wait()

    def _out(e, t):
        cvec = plsc.load_gather(cntv, [jnp.full((LANES,), e, _i32)])
        flag = (cvec > 0).astype(_f32)
        inv = jnp.float32(1.0) / jnp.maximum(cvec, jnp.float32(1.0))
        for j in range(D // LANES):
            sl = pl.ds(LANES * j, LANES)
            outv[e, sl] = (sumsv[e, sl] * inv + featv[e, sl]) * flag
        return t
    lax.fori_loop(0, EPT, _out, 0)

    def _drain_zero(out_ref, chunks):
        for _, nchunk in chunks:
            pltpu.make_async_copy(zbuf.at[pl.ds(0, nchunk)],
                                  out_ref.at[pl.ds(0, nchunk)], s_zero).wait()

    @pl.when(is0)
    def _():
        _drain_zero(out_u, _C0)

    @pl.when(is1)
    def _():
        _drain_zero(out_n, _C1)

    plsc.subcore_barrier()

    @pl.when(is0)
    def _():
        pltpu.sync_copy(outv, out_u.at[idxo])

    @pl.when(is1)
    def _():
        d9 = pltpu.async_copy(outv, out_n.at[idxo], s_c)
        d9.wait()


_gcn_sc = functools.partial(
    pl.kernel,
    out_type=(jax.ShapeDtypeStruct((N, D), _f32),
              jax.ShapeDtypeStruct((N, D), _f32)),
    mesh=_mesh,
    compiler_params=pltpu.CompilerParams(
        needs_layout_passes=False,
        skip_device_barrier=True,
        disable_bounds_checks=True,
        disable_semaphore_checks=True,
    ),
    scratch_types=[
        pltpu.VMEM((ZROWS, D), _f32),
        pltpu.VMEM((EPT, D), _f32),
        pltpu.VMEM((EPT, D), _f32),
        pltpu.VMEM((EPT, D), _f32),
        pltpu.VMEM((EPT, D), _f32),
        pltpu.VMEM((EPT, D), _f32),
        pltpu.VMEM((2, NSLOTS), _i32),
        pltpu.VMEM((EPT,), _i32),
        pltpu.VMEM((EPT,), _i32),
        pltpu.VMEM((EPT,), _f32),
        pltpu.VMEM((EPT,), _f32),
        pltpu.VMEM_SHARED((N, D), _f32),
        pltpu.VMEM_SHARED((N,), _f32),
        pltpu.SemaphoreType.DMA,
        pltpu.SemaphoreType.DMA,
        pltpu.SemaphoreType.DMA,
        pltpu.SemaphoreType.DMA,
    ],
)(_body)


def kernel(user_feat, news_feat, edge_feat, edge_index):
    return _gcn_sc(user_feat, news_feat, edge_feat, edge_index)
